# baseline (speedup 1.0000x reference)
import functools

import numpy as np
import jax
import jax.numpy as jnp
from jax.experimental import pallas as pl
from jax.experimental.pallas import tpu as pltpu

DIM = 384
NH = 12
HD = DIM // NH
K = 7
NE = 8
HID = 768
B, H, W = 2, 32, 32
NSTRIP = 4
QR = 8
KR = 16
QT = QR * W
KT = KR * W
T = H * W

_STARTS = np.clip(np.arange(H) - K // 2, 0, H - K)
_KS = np.array([min(max(8 * s - 4, 0), H - KR) for s in range(NSTRIP)])


def _bias_tables(rpb):
    s_ = np.arange(NSTRIP)[:, None, None]
    i_ = np.arange(QR)[None, :, None]
    j_ = np.arange(KR)[None, None, :]
    qr = 8 * s_ + i_
    kr = _KS[:, None, None] + j_
    qr, kr = np.broadcast_arrays(qr, kr)
    dr = kr - qr + (K - 1)
    rvalid = (kr >= _STARTS[qr]) & (kr < _STARTS[qr] + K)
    qc = np.arange(W)[:, None]
    kc = np.arange(W)[None, :]
    dc = kc - qc + (K - 1)
    cvalid = (kc >= _STARTS[qc]) & (kc < _STARTS[qc] + K)
    rel_r = np.clip(dr, 0, 2 * K - 2)
    rel_c = np.clip(dc, 0, 2 * K - 2)
    tab = rpb[:, rel_r[:, :, :, None, None],
              rel_c[None, None, None, :, :]]
    valid = (rvalid[:, :, :, None, None] & cvalid[None, None, None])
    tab = jnp.where(valid[None], tab, -1e9)
    tab = jnp.transpose(tab, (1, 0, 2, 4, 3, 5))
    return tab.reshape(NSTRIP, NH, QT, KT)


def _gelu(x):
    return 0.5 * x * (1.0 + jax.lax.erf(x * 0.7071067811865476))


def _ln(x, g, b):
    m = jnp.mean(x, axis=-1, keepdims=True)
    v = jnp.mean((x - m) ** 2, axis=-1, keepdims=True)
    return (x - m) * jax.lax.rsqrt(v + 1e-5) * g + b


def _attn_body(x_ref, ln1g_ref, ln1b_ref, wqkv_ref, bqkv_ref, bias_ref,
               wproj_ref, bproj_ref, ln2g_ref, ln2b_ref, wr_ref, br_ref,
               xout_ref, y_ref, ti_ref, tv_ref, qkv_scr):
    s = pl.program_id(0)
    ks = jnp.clip(8 * s - 4, 0, H - KR)
    xk = x_ref[0, pl.ds(ks, KR)].reshape(KT, DIM)
    xn = _ln(xk, ln1g_ref[...], ln1b_ref[...])
    qkv = jnp.dot(xn, wqkv_ref[...],
                  preferred_element_type=jnp.float32) + bqkv_ref[...]
    qo = 8 * s - ks
    qkv_scr[...] = qkv
    qrows = qkv_scr[pl.ds(qo * W, QT), :]
    scale = float(HD) ** -0.5
    outs = []
    for h in range(NH):
        qh = qrows[:, h * HD:(h + 1) * HD] * scale
        kh = qkv[:, DIM + h * HD:DIM + (h + 1) * HD]
        vh = qkv[:, 2 * DIM + h * HD:2 * DIM + (h + 1) * HD]
        sc = jax.lax.dot_general(qh, kh, (((1,), (1,)), ((), ())),
                                 preferred_element_type=jnp.float32)
        sc = sc + bias_ref[0, h]
        mx = jnp.max(sc, axis=-1, keepdims=True)
        p = jnp.exp(sc - mx)
        den = jnp.sum(p, axis=-1, keepdims=True)
        oh = jnp.dot(p, vh, preferred_element_type=jnp.float32) / den
        outs.append(oh)
    att = jnp.concatenate(outs, axis=1)
    proj = jnp.dot(att, wproj_ref[...],
                   preferred_element_type=jnp.float32) + bproj_ref[...]
    xa = x_ref[0, pl.ds(8 * s, QR)].reshape(QT, DIM) + proj
    xout_ref[0] = xa
    y = _ln(xa, ln2g_ref[...], ln2b_ref[...])
    y_ref[0] = y
    logits = jnp.dot(y, wr_ref[...],
                     preferred_element_type=jnp.float32) + br_ref[...]
    iota8 = jax.lax.broadcasted_iota(jnp.int32, (QT, NE), 1)
    m1 = jnp.max(logits, axis=-1, keepdims=True)
    i1 = jnp.min(jnp.where(logits >= m1, iota8, NE), axis=-1)
    l2m = jnp.where(iota8 == i1[:, None], -jnp.inf, logits)
    m2 = jnp.max(l2m, axis=-1, keepdims=True)
    i2 = jnp.min(jnp.where(l2m >= m2, iota8, NE), axis=-1)
    t = jnp.exp(m2[:, 0] - m1[:, 0])
    v1 = 1.0 / (1.0 + t)
    v2 = t / (1.0 + t)
    zi = jnp.zeros((NE - 2, QT), jnp.int32)
    zv = jnp.zeros((NE - 2, QT), jnp.float32)
    ti_ref[0] = jnp.concatenate([i1[None], i2[None], zi], axis=0)
    tv_ref[0] = jnp.concatenate([v1[None], v2[None], zv], axis=0)


def _moe_body(y_ref, xa_ref, ti_ref, tv_ref, we1_ref, be1_ref, we2_ref,
              be2_ref, ws1_ref, bs1_ref, ws2_ref, bs2_ref, out_ref):
    y = y_ref[0]
    i1 = ti_ref[0, 0, :]
    i2 = ti_ref[0, 1, :]
    v1 = tv_ref[0, 0, :]
    v2 = tv_ref[0, 1, :]
    hs = jnp.dot(y, ws1_ref[...],
                 preferred_element_type=jnp.float32) + bs1_ref[...]
    acc = jnp.dot(_gelu(hs), ws2_ref[...],
                  preferred_element_type=jnp.float32) + bs2_ref[...]
    for e in range(NE):
        h = jnp.dot(y, we1_ref[e],
                    preferred_element_type=jnp.float32) + be1_ref[e]
        eo = jnp.dot(_gelu(h), we2_ref[e],
                     preferred_element_type=jnp.float32) + be2_ref[e]
        ce = v1 * (i1 == e).astype(jnp.float32) + \
            v2 * (i2 == e).astype(jnp.float32)
        acc = acc + ce[:, None] * eo
    out_ref[0] = xa_ref[0] + acc


def kernel(x, ln1_g, ln1_b, ln2_g, ln2_b, W_qkv, b_qkv, rpb, W_proj, b_proj,
           W_r, b_r, W_e1, b_e1, W_e2, b_e2, W_s1, b_s1, W_s2, b_s2):
    bias_tab = _bias_tables(rpb)

    full = lambda *shape: pl.BlockSpec(shape, lambda s, b: (0,) * len(shape))
    attn_out = pl.pallas_call(
        _attn_body,
        grid=(NSTRIP, B),
        in_specs=[
            pl.BlockSpec((1, H, W, DIM), lambda s, b: (b, 0, 0, 0)),
            full(DIM), full(DIM),
            full(DIM, 3 * DIM), full(3 * DIM),
            pl.BlockSpec((1, NH, QT, KT), lambda s, b: (s, 0, 0, 0)),
            full(DIM, DIM), full(DIM),
            full(DIM), full(DIM),
            full(DIM, NE), full(NE),
        ],
        out_specs=[
            pl.BlockSpec((1, QT, DIM), lambda s, b: (b, s, 0)),
            pl.BlockSpec((1, QT, DIM), lambda s, b: (b, s, 0)),
            pl.BlockSpec((1, NE, QT), lambda s, b: (b, 0, s)),
            pl.BlockSpec((1, NE, QT), lambda s, b: (b, 0, s)),
        ],
        out_shape=[
            jax.ShapeDtypeStruct((B, T, DIM), jnp.float32),
            jax.ShapeDtypeStruct((B, T, DIM), jnp.float32),
            jax.ShapeDtypeStruct((B, NE, T), jnp.int32),
            jax.ShapeDtypeStruct((B, NE, T), jnp.float32),
        ],
        scratch_shapes=[pltpu.VMEM((KT, 3 * DIM), jnp.float32)],
    )(x, ln1_g, ln1_b, W_qkv, b_qkv, bias_tab, W_proj, b_proj,
      ln2_g, ln2_b, W_r, b_r)
    xa, y, ti, tv = attn_out

    fullm = lambda *shape: pl.BlockSpec(shape, lambda i: (0,) * len(shape))
    out = pl.pallas_call(
        _moe_body,
        grid=(B * NSTRIP,),
        in_specs=[
            pl.BlockSpec((1, QT, DIM), lambda i: (i // NSTRIP, i % NSTRIP, 0)),
            pl.BlockSpec((1, QT, DIM), lambda i: (i // NSTRIP, i % NSTRIP, 0)),
            pl.BlockSpec((1, NE, QT), lambda i: (i // NSTRIP, 0, i % NSTRIP)),
            pl.BlockSpec((1, NE, QT), lambda i: (i // NSTRIP, 0, i % NSTRIP)),
            fullm(NE, DIM, HID), fullm(NE, HID),
            fullm(NE, HID, DIM), fullm(NE, DIM),
            fullm(DIM, HID), fullm(HID),
            fullm(HID, DIM), fullm(DIM),
        ],
        out_specs=pl.BlockSpec((1, QT, DIM),
                               lambda i: (i // NSTRIP, i % NSTRIP, 0)),
        out_shape=jax.ShapeDtypeStruct((B, T, DIM), jnp.float32),
    )(y, xa, ti, tv, W_e1, b_e1, W_e2, b_e2, W_s1, b_s1, W_s2, b_s2)
    return out.reshape(B, H, W, DIM)


# speedup vs baseline: 28.0009x; 28.0009x over previous
import functools

import numpy as np
import jax
import jax.numpy as jnp
from jax.experimental import pallas as pl
from jax.experimental.pallas import tpu as pltpu

DIM = 384
NH = 12
HD = DIM // NH
K = 7
NE = 8
HID = 768
B, H, W = 2, 32, 32
NSTRIP = 4
QR = 8
KR = 16
QT = QR * W
KT = KR * W
T = H * W

_STARTS = np.clip(np.arange(H) - K // 2, 0, H - K)
_KS = np.array([min(max(8 * s - 4, 0), H - KR) for s in range(NSTRIP)])


def _onehot_tables():
    s_ = np.arange(NSTRIP)[:, None, None]
    i_ = np.arange(QR)[None, :, None]
    j_ = np.arange(KR)[None, None, :]
    qr = 8 * s_ + i_
    kr = _KS[:, None, None] + j_
    qr, kr = np.broadcast_arrays(qr, kr)
    dr = kr - qr + (K - 1)
    rvalid = (kr >= _STARTS[qr]) & (kr < _STARTS[qr] + K)
    ridx = np.where(rvalid, np.clip(dr, 0, 2 * K - 2), 2 * K - 1)
    ohr = np.zeros((NSTRIP, QR, KR, 2 * K), np.float32)
    np.put_along_axis(ohr, ridx[..., None], 1.0, axis=-1)
    qc = np.arange(W)[:, None]
    kc = np.arange(W)[None, :]
    dc = kc - qc + (K - 1)
    cvalid = (kc >= _STARTS[qc]) & (kc < _STARTS[qc] + K)
    cidx = np.where(cvalid, np.clip(dc, 0, 2 * K - 2), 2 * K - 1)
    ohc = np.zeros((W, W, 2 * K), np.float32)
    np.put_along_axis(ohc, cidx[..., None], 1.0, axis=-1)
    return ohr, ohc


_OHR, _OHC = _onehot_tables()


def _bias_tables(rpb):
    rpbe = jnp.full((NH, 2 * K, 2 * K), -1e9, jnp.float32)
    rpbe = rpbe.at[:, :2 * K - 1, :2 * K - 1].set(rpb)
    t2 = jnp.einsum('huv,qkv->huqk', rpbe, jnp.asarray(_OHC))
    tab = jnp.einsum('siju,huqk->shiqjk', jnp.asarray(_OHR), t2)
    return tab.reshape(NSTRIP, NH, QT, KT)


def _gelu(x):
    return 0.5 * x * (1.0 + jax.lax.erf(x * 0.7071067811865476))


def _ln(x, g, b):
    m = jnp.mean(x, axis=-1, keepdims=True)
    v = jnp.mean((x - m) ** 2, axis=-1, keepdims=True)
    return (x - m) * jax.lax.rsqrt(v + 1e-5) * g + b


def _attn_body(x_ref, ln1g_ref, ln1b_ref, wqkv_ref, bqkv_ref, bias_ref,
               wproj_ref, bproj_ref, ln2g_ref, ln2b_ref, wr_ref, br_ref,
               xout_ref, y_ref, ti_ref, tv_ref, qkv_scr):
    s = pl.program_id(0)
    ks = jnp.clip(8 * s - 4, 0, H - KR)
    xk = x_ref[0, pl.ds(ks, KR)].reshape(KT, DIM)
    xn = _ln(xk, ln1g_ref[...], ln1b_ref[...])
    qkv = jnp.dot(xn, wqkv_ref[...],
                  preferred_element_type=jnp.float32) + bqkv_ref[...]
    qo = 8 * s - ks
    qkv_scr[...] = qkv
    qrows = qkv_scr[pl.ds(qo * W, QT), :]
    scale = float(HD) ** -0.5
    outs = []
    for h in range(NH):
        qh = qrows[:, h * HD:(h + 1) * HD] * scale
        kh = qkv[:, DIM + h * HD:DIM + (h + 1) * HD]
        vh = qkv[:, 2 * DIM + h * HD:2 * DIM + (h + 1) * HD]
        sc = jax.lax.dot_general(qh, kh, (((1,), (1,)), ((), ())),
                                 preferred_element_type=jnp.float32)
        sc = sc + bias_ref[0, h]
        mx = jnp.max(sc, axis=-1, keepdims=True)
        p = jnp.exp(sc - mx)
        den = jnp.sum(p, axis=-1, keepdims=True)
        oh = jnp.dot(p, vh, preferred_element_type=jnp.float32) / den
        outs.append(oh)
    att = jnp.concatenate(outs, axis=1)
    proj = jnp.dot(att, wproj_ref[...],
                   preferred_element_type=jnp.float32) + bproj_ref[...]
    xa = x_ref[0, pl.ds(8 * s, QR)].reshape(QT, DIM) + proj
    xout_ref[0] = xa
    y = _ln(xa, ln2g_ref[...], ln2b_ref[...])
    y_ref[0] = y
    logits = jnp.dot(y, wr_ref[...],
                     preferred_element_type=jnp.float32) + br_ref[...]
    iota8 = jax.lax.broadcasted_iota(jnp.int32, (QT, NE), 1)
    m1 = jnp.max(logits, axis=-1, keepdims=True)
    i1 = jnp.min(jnp.where(logits >= m1, iota8, NE), axis=-1)
    l2m = jnp.where(iota8 == i1[:, None], -jnp.inf, logits)
    m2 = jnp.max(l2m, axis=-1, keepdims=True)
    i2 = jnp.min(jnp.where(l2m >= m2, iota8, NE), axis=-1)
    t = jnp.exp(m2[:, 0] - m1[:, 0])
    v1 = 1.0 / (1.0 + t)
    v2 = t / (1.0 + t)
    zi = jnp.zeros((NE - 2, QT), jnp.int32)
    zv = jnp.zeros((NE - 2, QT), jnp.float32)
    ti_ref[0] = jnp.concatenate([i1[None], i2[None], zi], axis=0)
    tv_ref[0] = jnp.concatenate([v1[None], v2[None], zv], axis=0)


def _moe_body(y_ref, xa_ref, ti_ref, tv_ref, we1_ref, be1_ref, we2_ref,
              be2_ref, ws1_ref, bs1_ref, ws2_ref, bs2_ref, out_ref):
    y = y_ref[0]
    i1 = ti_ref[0, 0, :]
    i2 = ti_ref[0, 1, :]
    v1 = tv_ref[0, 0, :]
    v2 = tv_ref[0, 1, :]
    hs = jnp.dot(y, ws1_ref[...],
                 preferred_element_type=jnp.float32) + bs1_ref[...]
    acc = jnp.dot(_gelu(hs), ws2_ref[...],
                  preferred_element_type=jnp.float32) + bs2_ref[...]
    for e in range(NE):
        h = jnp.dot(y, we1_ref[e],
                    preferred_element_type=jnp.float32) + be1_ref[e]
        eo = jnp.dot(_gelu(h), we2_ref[e],
                     preferred_element_type=jnp.float32) + be2_ref[e]
        ce = v1 * (i1 == e).astype(jnp.float32) + \
            v2 * (i2 == e).astype(jnp.float32)
        acc = acc + ce[:, None] * eo
    out_ref[0] = xa_ref[0] + acc


def kernel(x, ln1_g, ln1_b, ln2_g, ln2_b, W_qkv, b_qkv, rpb, W_proj, b_proj,
           W_r, b_r, W_e1, b_e1, W_e2, b_e2, W_s1, b_s1, W_s2, b_s2):
    bias_tab = _bias_tables(rpb)

    full = lambda *shape: pl.BlockSpec(shape, lambda s, b: (0,) * len(shape))
    attn_out = pl.pallas_call(
        _attn_body,
        grid=(NSTRIP, B),
        in_specs=[
            pl.BlockSpec((1, H, W, DIM), lambda s, b: (b, 0, 0, 0)),
            full(DIM), full(DIM),
            full(DIM, 3 * DIM), full(3 * DIM),
            pl.BlockSpec((1, NH, QT, KT), lambda s, b: (s, 0, 0, 0)),
            full(DIM, DIM), full(DIM),
            full(DIM), full(DIM),
            full(DIM, NE), full(NE),
        ],
        out_specs=[
            pl.BlockSpec((1, QT, DIM), lambda s, b: (b, s, 0)),
            pl.BlockSpec((1, QT, DIM), lambda s, b: (b, s, 0)),
            pl.BlockSpec((1, NE, QT), lambda s, b: (b, 0, s)),
            pl.BlockSpec((1, NE, QT), lambda s, b: (b, 0, s)),
        ],
        out_shape=[
            jax.ShapeDtypeStruct((B, T, DIM), jnp.float32),
            jax.ShapeDtypeStruct((B, T, DIM), jnp.float32),
            jax.ShapeDtypeStruct((B, NE, T), jnp.int32),
            jax.ShapeDtypeStruct((B, NE, T), jnp.float32),
        ],
        scratch_shapes=[pltpu.VMEM((KT, 3 * DIM), jnp.float32)],
    )(x, ln1_g, ln1_b, W_qkv, b_qkv, bias_tab, W_proj, b_proj,
      ln2_g, ln2_b, W_r, b_r)
    xa, y, ti, tv = attn_out

    fullm = lambda *shape: pl.BlockSpec(shape, lambda i: (0,) * len(shape))
    out = pl.pallas_call(
        _moe_body,
        grid=(B * NSTRIP,),
        in_specs=[
            pl.BlockSpec((1, QT, DIM), lambda i: (i // NSTRIP, i % NSTRIP, 0)),
            pl.BlockSpec((1, QT, DIM), lambda i: (i // NSTRIP, i % NSTRIP, 0)),
            pl.BlockSpec((1, NE, QT), lambda i: (i // NSTRIP, 0, i % NSTRIP)),
            pl.BlockSpec((1, NE, QT), lambda i: (i // NSTRIP, 0, i % NSTRIP)),
            fullm(NE, DIM, HID), fullm(NE, HID),
            fullm(NE, HID, DIM), fullm(NE, DIM),
            fullm(DIM, HID), fullm(HID),
            fullm(HID, DIM), fullm(DIM),
        ],
        out_specs=pl.BlockSpec((1, QT, DIM),
                               lambda i: (i // NSTRIP, i % NSTRIP, 0)),
        out_shape=jax.ShapeDtypeStruct((B, T, DIM), jnp.float32),
    )(y, xa, ti, tv, W_e1, b_e1, W_e2, b_e2, W_s1, b_s1, W_s2, b_s2)
    return out.reshape(B, H, W, DIM)


# speedup vs baseline: 34.8242x; 1.2437x over previous
import functools

import numpy as np
import jax
import jax.numpy as jnp
from jax.experimental import pallas as pl
from jax.experimental.pallas import tpu as pltpu

DIM = 384
NH = 12
HD = DIM // NH
K = 7
NE = 8
HID = 768
B, H, W = 2, 32, 32
NSTRIP = 4
QR = 8
KR = 16
QT = QR * W
KT = KR * W
T = H * W

_STARTS = np.clip(np.arange(H) - K // 2, 0, H - K)
_KS = np.array([min(max(8 * s - 4, 0), H - KR) for s in range(NSTRIP)])


def _onehot_tables():
    s_ = np.arange(NSTRIP)[:, None, None]
    i_ = np.arange(QR)[None, :, None]
    j_ = np.arange(KR)[None, None, :]
    qr = 8 * s_ + i_
    kr = _KS[:, None, None] + j_
    qr, kr = np.broadcast_arrays(qr, kr)
    dr = kr - qr + (K - 1)
    rvalid = (kr >= _STARTS[qr]) & (kr < _STARTS[qr] + K)
    ridx = np.where(rvalid, np.clip(dr, 0, 2 * K - 2), 2 * K - 1)
    ohr = np.zeros((NSTRIP, QR, KR, 2 * K), np.float32)
    np.put_along_axis(ohr, ridx[..., None], 1.0, axis=-1)
    assert np.array_equal(ohr[1], ohr[2])
    ohr = ohr[[0, 1, 3]]
    qc = np.arange(W)[:, None]
    kc = np.arange(W)[None, :]
    dc = kc - qc + (K - 1)
    cvalid = (kc >= _STARTS[qc]) & (kc < _STARTS[qc] + K)
    cidx = np.where(cvalid, np.clip(dc, 0, 2 * K - 2), 2 * K - 1)
    ohc = np.zeros((W, W, 2 * K), np.float32)
    np.put_along_axis(ohc, cidx[..., None], 1.0, axis=-1)
    return ohr, ohc


_OHR, _OHC = _onehot_tables()


def _bias_tables(rpb):
    rpbe = jnp.full((NH, 2 * K, 2 * K), -1e9, jnp.float32)
    rpbe = rpbe.at[:, :2 * K - 1, :2 * K - 1].set(rpb)
    t2 = jnp.einsum('huv,qkv->huqk', rpbe, jnp.asarray(_OHC))
    tab = jnp.einsum('siju,huqk->shiqjk', jnp.asarray(_OHR), t2)
    return tab.reshape(3, NH, QT, KT).astype(jnp.bfloat16)


def _gelu(x):
    return 0.5 * x * (1.0 + jax.lax.erf(x * 0.7071067811865476))


def _ln(x, g, b):
    m = jnp.mean(x, axis=-1, keepdims=True)
    v = jnp.mean((x - m) ** 2, axis=-1, keepdims=True)
    return (x - m) * jax.lax.rsqrt(v + 1e-5) * g + b


def _attn_body(x_ref, ln1g_ref, ln1b_ref, wqkv_ref, bqkv_ref, bias_ref,
               wproj_ref, bproj_ref, ln2g_ref, ln2b_ref, wr_ref, br_ref,
               xout_ref, y_ref, ti_ref, tv_ref, qkv_scr):
    s = pl.program_id(0)
    ks = jnp.clip(8 * s - 4, 0, H - KR)
    xk = x_ref[0, pl.ds(ks, KR)].reshape(KT, DIM)
    xn = _ln(xk, ln1g_ref[...], ln1b_ref[...])
    qkv = jnp.dot(xn, wqkv_ref[...],
                  preferred_element_type=jnp.float32) + bqkv_ref[...]
    qo = 8 * s - ks
    qkv_scr[...] = qkv
    qrows = qkv_scr[pl.ds(qo * W, QT), :]
    scale = float(HD) ** -0.5
    outs = []
    for h in range(NH):
        qh = qrows[:, h * HD:(h + 1) * HD] * scale
        kh = qkv[:, DIM + h * HD:DIM + (h + 1) * HD]
        vh = qkv[:, 2 * DIM + h * HD:2 * DIM + (h + 1) * HD]
        sc = jax.lax.dot_general(qh, kh, (((1,), (1,)), ((), ())),
                                 preferred_element_type=jnp.float32)
        sc = sc + bias_ref[0, h].astype(jnp.float32)
        mx = jnp.max(sc, axis=-1, keepdims=True)
        p = jnp.exp(sc - mx)
        den = jnp.sum(p, axis=-1, keepdims=True)
        oh = jnp.dot(p, vh, preferred_element_type=jnp.float32) / den
        outs.append(oh)
    att = jnp.concatenate(outs, axis=1)
    proj = jnp.dot(att, wproj_ref[...],
                   preferred_element_type=jnp.float32) + bproj_ref[...]
    xa = x_ref[0, pl.ds(8 * s, QR)].reshape(QT, DIM) + proj
    xout_ref[0] = xa
    y = _ln(xa, ln2g_ref[...], ln2b_ref[...])
    y_ref[0] = y
    logits = jnp.dot(y, wr_ref[...],
                     preferred_element_type=jnp.float32) + br_ref[...]
    iota8 = jax.lax.broadcasted_iota(jnp.int32, (QT, NE), 1)
    m1 = jnp.max(logits, axis=-1, keepdims=True)
    i1 = jnp.min(jnp.where(logits >= m1, iota8, NE), axis=-1)
    l2m = jnp.where(iota8 == i1[:, None], -jnp.inf, logits)
    m2 = jnp.max(l2m, axis=-1, keepdims=True)
    i2 = jnp.min(jnp.where(l2m >= m2, iota8, NE), axis=-1)
    t = jnp.exp(m2[:, 0] - m1[:, 0])
    v1 = 1.0 / (1.0 + t)
    v2 = t / (1.0 + t)
    zi = jnp.zeros((NE - 2, QT), jnp.int32)
    zv = jnp.zeros((NE - 2, QT), jnp.float32)
    ti_ref[0] = jnp.concatenate([i1[None], i2[None], zi], axis=0)
    tv_ref[0] = jnp.concatenate([v1[None], v2[None], zv], axis=0)


def _moe_body(y_ref, xa_ref, ti_ref, tv_ref, we1_ref, be1_ref, we2_ref,
              be2_ref, ws1_ref, bs1_ref, ws2_ref, bs2_ref, out_ref):
    y = y_ref[0]
    i1 = ti_ref[0, 0, :]
    i2 = ti_ref[0, 1, :]
    v1 = tv_ref[0, 0, :]
    v2 = tv_ref[0, 1, :]
    hs = jnp.dot(y, ws1_ref[...],
                 preferred_element_type=jnp.float32) + bs1_ref[...]
    acc = jnp.dot(_gelu(hs), ws2_ref[...],
                  preferred_element_type=jnp.float32) + bs2_ref[...]
    for e in range(NE):
        h = jnp.dot(y, we1_ref[e],
                    preferred_element_type=jnp.float32) + be1_ref[e]
        eo = jnp.dot(_gelu(h), we2_ref[e],
                     preferred_element_type=jnp.float32) + be2_ref[e]
        ce = v1 * (i1 == e).astype(jnp.float32) + \
            v2 * (i2 == e).astype(jnp.float32)
        acc = acc + ce[:, None] * eo
    out_ref[0] = xa_ref[0] + acc


def kernel(x, ln1_g, ln1_b, ln2_g, ln2_b, W_qkv, b_qkv, rpb, W_proj, b_proj,
           W_r, b_r, W_e1, b_e1, W_e2, b_e2, W_s1, b_s1, W_s2, b_s2):
    bias_tab = _bias_tables(rpb)

    full = lambda *shape: pl.BlockSpec(shape, lambda s, b: (0,) * len(shape))
    attn_out = pl.pallas_call(
        _attn_body,
        grid=(NSTRIP, B),
        in_specs=[
            pl.BlockSpec((1, H, W, DIM), lambda s, b: (b, 0, 0, 0)),
            full(DIM), full(DIM),
            full(DIM, 3 * DIM), full(3 * DIM),
            pl.BlockSpec((1, NH, QT, KT),
                         lambda s, b: ((s > 0).astype(jnp.int32)
                                      + (s == 3).astype(jnp.int32), 0, 0, 0)),
            full(DIM, DIM), full(DIM),
            full(DIM), full(DIM),
            full(DIM, NE), full(NE),
        ],
        out_specs=[
            pl.BlockSpec((1, QT, DIM), lambda s, b: (b, s, 0)),
            pl.BlockSpec((1, QT, DIM), lambda s, b: (b, s, 0)),
            pl.BlockSpec((1, NE, QT), lambda s, b: (b, 0, s)),
            pl.BlockSpec((1, NE, QT), lambda s, b: (b, 0, s)),
        ],
        out_shape=[
            jax.ShapeDtypeStruct((B, T, DIM), jnp.float32),
            jax.ShapeDtypeStruct((B, T, DIM), jnp.float32),
            jax.ShapeDtypeStruct((B, NE, T), jnp.int32),
            jax.ShapeDtypeStruct((B, NE, T), jnp.float32),
        ],
        scratch_shapes=[pltpu.VMEM((KT, 3 * DIM), jnp.float32)],
    )(x, ln1_g, ln1_b, W_qkv, b_qkv, bias_tab, W_proj, b_proj,
      ln2_g, ln2_b, W_r, b_r)
    xa, y, ti, tv = attn_out

    fullm = lambda *shape: pl.BlockSpec(shape, lambda i: (0,) * len(shape))
    out = pl.pallas_call(
        _moe_body,
        grid=(B * NSTRIP,),
        in_specs=[
            pl.BlockSpec((1, QT, DIM), lambda i: (i // NSTRIP, i % NSTRIP, 0)),
            pl.BlockSpec((1, QT, DIM), lambda i: (i // NSTRIP, i % NSTRIP, 0)),
            pl.BlockSpec((1, NE, QT), lambda i: (i // NSTRIP, 0, i % NSTRIP)),
            pl.BlockSpec((1, NE, QT), lambda i: (i // NSTRIP, 0, i % NSTRIP)),
            fullm(NE, DIM, HID), fullm(NE, HID),
            fullm(NE, HID, DIM), fullm(NE, DIM),
            fullm(DIM, HID), fullm(HID),
            fullm(HID, DIM), fullm(DIM),
        ],
        out_specs=pl.BlockSpec((1, QT, DIM),
                               lambda i: (i // NSTRIP, i % NSTRIP, 0)),
        out_shape=jax.ShapeDtypeStruct((B, T, DIM), jnp.float32),
    )(y, xa, ti, tv, W_e1, b_e1, W_e2, b_e2, W_s1, b_s1, W_s2, b_s2)
    return out.reshape(B, H, W, DIM)
